# baseline (speedup 1.0000x reference)
import functools

import jax
import jax.numpy as jnp
from jax import lax
from jax.experimental import pallas as pl
from jax.experimental.pallas import tpu as pltpu
from jax.experimental.pallas import tpu_sc as plsc

USERS = 10000
ITEMS = 40000
N = USERS + ITEMS
E = 800000
D = 64
B = 1024
N_LAYERS = 3

NC = 2
NS = 16
HALF = N // NC
ZCH = 80
NZCH = (HALF + ZCH - 1) // ZCH
ACC_ROWS = NZCH * ZCH
DUMMY = HALF

SUB = 400
NCHUNK = E // SUB // NS
EROW = 3 * SUB
DR = 200
NDR = HALF // DR


def _propagate_body(edata_hbm, emb_hbm, out_hbm,
                    acc, ebuf, dloc, gbuf, isem, gsem, ssem):
    c = lax.axis_index("c")
    s = lax.axis_index("s")
    base = c * HALF

    def zfill(r, carry):
        for cc in range(2):
            gbuf[0, r, pl.ds(cc * 32, 32)] = jnp.zeros((32,), jnp.bfloat16)
        return carry
    lax.fori_loop(0, ZCH, zfill, 0)

    def zcopy(j, carry):
        idx = s + j * NS

        @pl.when(idx < NZCH)
        def _():
            pltpu.sync_copy(gbuf.at[0, pl.ds(0, ZCH)],
                            acc.at[pl.ds(idx * ZCH, ZCH)])
        return carry
    lax.fori_loop(0, (NZCH + NS - 1) // NS, zcopy, 0)
    plsc.subcore_barrier()

    def _idx_copy(i, e):
        eoff = (s * NCHUNK + i) * EROW
        return (edata_hbm.at[pl.ds(eoff, EROW)], ebuf.at[e])

    for p in range(3):
        sp, dp = _idx_copy(p, p)
        pltpu.async_copy(sp, dp, isem.at[p])

    def _gather_pair(e, b):
        return (emb_hbm.at[ebuf.at[e, pl.ds(0, SUB)]], gbuf.at[b])

    def chunk(i, carry):
        b = lax.rem(i, 2)
        nb = 1 - b
        e = lax.rem(i, 4)

        @pl.when(i >= 1)
        def _():
            ep = lax.rem(i + 3, 4)
            gsr, gdr = _gather_pair(ep, nb)
            pltpu.make_async_copy(gsr, gdr, gsem.at[nb]).wait()

            def scale(g, carry):
                wv16 = plsc.bitcast(ebuf[ep, pl.ds(2 * SUB + g * 16, 16)],
                                    jnp.float32)
                for lane in range(16):
                    wf = jnp.full((16,), wv16[lane], jnp.float32)
                    wsp = plsc.pack(wf, wf,
                                    format=plsc.PackFormat.INTERLEAVED)
                    jj = g * 16 + lane
                    for cc in range(2):
                        gbuf[nb, jj, pl.ds(cc * 32, 32)] = (
                            gbuf[nb, jj, pl.ds(cc * 32, 32)] * wsp)
                return carry
            lax.fori_loop(0, SUB // 16, scale, 0)
            pltpu.async_copy(gbuf.at[nb], acc.at[dloc.at[nb]], ssem.at[nb],
                             add=True)

        @pl.when(i < NCHUNK)
        def _():
            sr, dr = _idx_copy(i, e)
            pltpu.make_async_copy(sr, dr, isem.at[e]).wait()

            @pl.when(i + 3 < NCHUNK)
            def _():
                sr2, dr2 = _idx_copy(i + 3, lax.rem(i + 3, 4))
                pltpu.async_copy(sr2, dr2, isem.at[lax.rem(i + 3, 4)])

            @pl.when(i >= 2)
            def _():
                pltpu.make_async_copy(gbuf.at[b], acc.at[dloc.at[b]],
                                      ssem.at[b]).wait()

            gsr, gdr = _gather_pair(e, b)
            pltpu.async_copy(gsr, gdr, gsem.at[b])

            for t in range(SUB // 16):
                d = ebuf[e, pl.ds(SUB + t * 16, 16)]
                loc = d - base
                ok = (loc >= 0) & (loc < HALF)
                dloc[b, pl.ds(t * 16, 16)] = jnp.where(ok, loc, DUMMY)
        return carry
    lax.fori_loop(0, NCHUNK + 1, chunk, 0)
    for b in range(2):
        pltpu.make_async_copy(gbuf.at[b], acc.at[dloc.at[b]],
                              ssem.at[b]).wait()
    plsc.subcore_barrier()

    def drain(j, carry):
        idx = s + j * NS

        @pl.when(idx < NDR)
        def _():
            pltpu.sync_copy(acc.at[pl.ds(idx * DR, DR)],
                            out_hbm.at[pl.ds(base + idx * DR, DR)])
        return carry
    lax.fori_loop(0, (NDR + NS - 1) // NS, drain, 0)


def _user_gather_body(e1, e2, e3, uid_hbm, out_hbm, idx, r1, r2, r3, sem):
    c = lax.axis_index("c")
    s = lax.axis_index("s")
    w = s * NC + c
    rows = B // (NC * NS)
    base = w * rows
    pltpu.sync_copy(uid_hbm.at[pl.ds(base, rows)], idx)
    d1 = pltpu.async_copy(e1.at[idx], r1, sem)
    d2 = pltpu.async_copy(e2.at[idx], r2, sem)
    d3 = pltpu.async_copy(e3.at[idx], r3, sem)
    d1.wait(); d2.wait(); d3.wait()
    for r in range(rows):
        for cc in range(2):
            sl = pl.ds(cc * 32, 32)
            r1[r, sl] = r1[r, sl] + r2[r, sl] + r3[r, sl]
    pltpu.sync_copy(r1, out_hbm.at[pl.ds(base, rows)])


def _mm_body(u_ref, i1_ref, i2_ref, i3_ref, o_ref):
    its = (i1_ref[...].astype(jnp.float32) + i2_ref[...].astype(jnp.float32)
           + i3_ref[...].astype(jnp.float32)).astype(jnp.bfloat16)
    logits = lax.dot_general(u_ref[...], its, (((1,), (1,)), ((), ())),
                             preferred_element_type=jnp.float32)
    x = logits * (1.0 / 9.0)
    o_ref[...] = 1.0 / (1.0 + jnp.exp(-x))


_MESH = plsc.VectorSubcoreMesh(core_axis_name="c", subcore_axis_name="s")

_SC_PARAMS = pltpu.CompilerParams(use_tc_tiling_on_sc=False,
                                  needs_layout_passes=False)

_propagate = pl.kernel(
    _propagate_body,
    out_type=jax.ShapeDtypeStruct((N, D), jnp.bfloat16),
    mesh=_MESH,
    compiler_params=_SC_PARAMS,
    scratch_types=[
        pltpu.VMEM_SHARED((ACC_ROWS, D), jnp.bfloat16),
        pltpu.VMEM((4, EROW), jnp.int32),
        pltpu.VMEM((2, SUB), jnp.int32),
        pltpu.VMEM((2, SUB, D), jnp.bfloat16),
        pltpu.SemaphoreType.DMA((4,)),
        pltpu.SemaphoreType.DMA((2,)),
        pltpu.SemaphoreType.DMA((2,)),
    ],
)

_user_gather = pl.kernel(
    _user_gather_body,
    out_type=jax.ShapeDtypeStruct((B, D), jnp.bfloat16),
    mesh=_MESH,
    compiler_params=_SC_PARAMS,
    scratch_types=[
        pltpu.VMEM((B // (NC * NS),), jnp.int32),
        pltpu.VMEM((B // (NC * NS), D), jnp.bfloat16),
        pltpu.VMEM((B // (NC * NS), D), jnp.bfloat16),
        pltpu.VMEM((B // (NC * NS), D), jnp.bfloat16),
        pltpu.SemaphoreType.DMA,
    ],
)

_IB = 1024

_matmul = pl.pallas_call(
    _mm_body,
    grid=(pl.cdiv(ITEMS, _IB),),
    in_specs=[
        pl.BlockSpec((B, D), lambda i: (0, 0)),
        pl.BlockSpec((_IB, D), lambda i: (i, 0)),
        pl.BlockSpec((_IB, D), lambda i: (i, 0)),
        pl.BlockSpec((_IB, D), lambda i: (i, 0)),
    ],
    out_specs=pl.BlockSpec((B, _IB), lambda i: (0, i)),
    out_shape=jax.ShapeDtypeStruct((B, ITEMS), jnp.float32),
)


def kernel(user_ids, edge_index, edge_weight, user_table, item_table):
    nch = E // SUB
    edata = jnp.concatenate([
        edge_index[0].reshape(nch, 1, SUB),
        edge_index[1].reshape(nch, 1, SUB),
        lax.bitcast_convert_type(edge_weight, jnp.int32).reshape(nch, 1, SUB),
    ], axis=1).reshape(-1)
    emb = jnp.concatenate([user_table, item_table],
                          axis=0).astype(jnp.bfloat16)

    e1 = _propagate(edata, emb)
    e2 = _propagate(edata, e1)
    e3 = _propagate(edata, e2)

    u_sum = _user_gather(e1, e2, e3, user_ids)
    return _matmul(u_sum, e1[USERS:], e2[USERS:], e3[USERS:])


# speedup vs baseline: 1.0699x; 1.0699x over previous
import functools

import jax
import jax.numpy as jnp
from jax import lax
from jax.experimental import pallas as pl
from jax.experimental.pallas import tpu as pltpu
from jax.experimental.pallas import tpu_sc as plsc

USERS = 10000
ITEMS = 40000
N = USERS + ITEMS
E = 800000
D = 64
B = 1024
N_LAYERS = 3

NC = 2
NS = 16
HALF = N // NC
ZCH = 80
NZCH = (HALF + ZCH - 1) // ZCH
ACC_ROWS = NZCH * ZCH
DUMMY = HALF

K2 = 80
ROWS_PER_TILE = E // K2 // NS
CH = 5
NCHUNK = ROWS_PER_TILE // CH
DR = 200
NDR = HALF // DR


def _propagate_body(src_hbm, dst_hbm, w_hbm, emb_hbm, out_hbm,
                    acc, sidx, didx, dloc, wbuf, gbuf, isem, gsem, ssem):
    c = lax.axis_index("c")
    s = lax.axis_index("s")
    base = c * HALF

    def zfill(r, carry):
        for q in range(CH):
            for cc in range(2):
                gbuf[q, r, pl.ds(cc * 32, 32)] = jnp.zeros((32,),
                                                           jnp.bfloat16)
        return carry
    lax.fori_loop(0, K2, zfill, 0)

    def zcopy(j, carry):
        idx = s + j * NS

        @pl.when(idx < NZCH)
        def _():
            pltpu.sync_copy(gbuf.at[0], acc.at[pl.ds(idx * ZCH, ZCH)])
        return carry
    lax.fori_loop(0, (NZCH + NS - 1) // NS, zcopy, 0)
    plsc.subcore_barrier()

    def _idx_copies(i, b):
        eoff = (s * ROWS_PER_TILE // CH + i) * (CH * K2)
        return [
            (src_hbm.at[pl.ds(eoff, CH * K2)], sidx.at[b]),
            (dst_hbm.at[pl.ds(eoff, CH * K2)], didx.at[b]),
            (w_hbm.at[pl.ds(eoff, CH * K2)], wbuf.at[b]),
        ]

    for sr, dr in _idx_copies(0, 0):
        pltpu.async_copy(sr, dr, isem)

    def chunk(i, carry):
        b = lax.rem(i, 2)
        for sr, dr in _idx_copies(i, b):
            pltpu.make_async_copy(sr, dr, isem).wait()

        @pl.when(i + 1 < NCHUNK)
        def _():
            for sr, dr in _idx_copies(i + 1, 1 - b):
                pltpu.async_copy(sr, dr, isem)

        for q in range(CH):
            @pl.when(i > 0)
            def _(q=q):
                pltpu.make_async_copy(gbuf.at[q], acc.at[dloc.at[q]],
                                      ssem.at[q]).wait()
            pltpu.async_copy(emb_hbm.at[sidx.at[b, pl.ds(q * K2, K2)]],
                             gbuf.at[q], gsem.at[q])

        for q in range(CH):
            for t in range(K2 // 16):
                d = didx[b, pl.ds(q * K2 + t * 16, 16)]
                loc = d - base
                ok = (loc >= 0) & (loc < HALF)
                dloc[q, pl.ds(t * 16, 16)] = jnp.where(ok, loc, DUMMY)

        for q in range(CH):
            pltpu.make_async_copy(emb_hbm.at[sidx.at[b, pl.ds(q * K2, K2)]],
                                  gbuf.at[q], gsem.at[q]).wait()

            def scale(g, carry, q=q):
                wv16 = wbuf[b, pl.ds(q * K2 + g * 16, 16)]
                for lane in range(16):
                    wf = jnp.full((16,), wv16[lane], jnp.float32)
                    wsp = plsc.pack(wf, wf,
                                    format=plsc.PackFormat.INTERLEAVED)
                    jj = g * 16 + lane
                    for cc in range(2):
                        gbuf[q, jj, pl.ds(cc * 32, 32)] = (
                            gbuf[q, jj, pl.ds(cc * 32, 32)] * wsp)
                return carry
            lax.fori_loop(0, K2 // 16, scale, 0)
            pltpu.async_copy(gbuf.at[q], acc.at[dloc.at[q]], ssem.at[q],
                             add=True)
        return carry
    lax.fori_loop(0, NCHUNK, chunk, 0)
    for q in range(CH):
        pltpu.make_async_copy(gbuf.at[q], acc.at[dloc.at[q]],
                              ssem.at[q]).wait()
    plsc.subcore_barrier()

    def drain(j, carry):
        idx = s + j * NS

        @pl.when(idx < NDR)
        def _():
            pltpu.sync_copy(acc.at[pl.ds(idx * DR, DR)],
                            out_hbm.at[pl.ds(base + idx * DR, DR)])
        return carry
    lax.fori_loop(0, (NDR + NS - 1) // NS, drain, 0)


def _user_gather_body(e1, e2, e3, uid_hbm, out_hbm, idx, r1, r2, r3, sem):
    c = lax.axis_index("c")
    s = lax.axis_index("s")
    w = s * NC + c
    rows = B // (NC * NS)
    base = w * rows
    pltpu.sync_copy(uid_hbm.at[pl.ds(base, rows)], idx)
    d1 = pltpu.async_copy(e1.at[idx], r1, sem)
    d2 = pltpu.async_copy(e2.at[idx], r2, sem)
    d3 = pltpu.async_copy(e3.at[idx], r3, sem)
    d1.wait(); d2.wait(); d3.wait()
    for r in range(rows):
        for cc in range(2):
            sl = pl.ds(cc * 32, 32)
            r1[r, sl] = r1[r, sl] + r2[r, sl] + r3[r, sl]
    pltpu.sync_copy(r1, out_hbm.at[pl.ds(base, rows)])


def _mm_body(u_ref, i1_ref, i2_ref, i3_ref, o_ref):
    its = (i1_ref[...].astype(jnp.float32) + i2_ref[...].astype(jnp.float32)
           + i3_ref[...].astype(jnp.float32)).astype(jnp.bfloat16)
    logits = lax.dot_general(u_ref[...], its, (((1,), (1,)), ((), ())),
                             preferred_element_type=jnp.float32)
    x = logits * (1.0 / 9.0)
    o_ref[...] = 1.0 / (1.0 + jnp.exp(-x))


_MESH = plsc.VectorSubcoreMesh(core_axis_name="c", subcore_axis_name="s")

_SC_PARAMS = pltpu.CompilerParams(use_tc_tiling_on_sc=False,
                                  needs_layout_passes=False)

_propagate = pl.kernel(
    _propagate_body,
    out_type=jax.ShapeDtypeStruct((N, D), jnp.bfloat16),
    mesh=_MESH,
    compiler_params=_SC_PARAMS,
    scratch_types=[
        pltpu.VMEM_SHARED((ACC_ROWS, D), jnp.bfloat16),
        pltpu.VMEM((2, CH * K2), jnp.int32),
        pltpu.VMEM((2, CH * K2), jnp.int32),
        pltpu.VMEM((CH, K2), jnp.int32),
        pltpu.VMEM((2, CH * K2), jnp.float32),
        pltpu.VMEM((CH, K2, D), jnp.bfloat16),
        pltpu.SemaphoreType.DMA,
        pltpu.SemaphoreType.DMA((CH,)),
        pltpu.SemaphoreType.DMA((CH,)),
    ],
)

_user_gather = pl.kernel(
    _user_gather_body,
    out_type=jax.ShapeDtypeStruct((B, D), jnp.bfloat16),
    mesh=_MESH,
    compiler_params=_SC_PARAMS,
    scratch_types=[
        pltpu.VMEM((B // (NC * NS),), jnp.int32),
        pltpu.VMEM((B // (NC * NS), D), jnp.bfloat16),
        pltpu.VMEM((B // (NC * NS), D), jnp.bfloat16),
        pltpu.VMEM((B // (NC * NS), D), jnp.bfloat16),
        pltpu.SemaphoreType.DMA,
    ],
)

_IB = 1024

_matmul = pl.pallas_call(
    _mm_body,
    grid=(pl.cdiv(ITEMS, _IB),),
    in_specs=[
        pl.BlockSpec((B, D), lambda i: (0, 0)),
        pl.BlockSpec((_IB, D), lambda i: (i, 0)),
        pl.BlockSpec((_IB, D), lambda i: (i, 0)),
        pl.BlockSpec((_IB, D), lambda i: (i, 0)),
    ],
    out_specs=pl.BlockSpec((B, _IB), lambda i: (0, i)),
    out_shape=jax.ShapeDtypeStruct((B, ITEMS), jnp.float32),
)


def kernel(user_ids, edge_index, edge_weight, user_table, item_table):
    src = edge_index[0]
    dst = edge_index[1]
    w = edge_weight
    emb = jnp.concatenate([user_table, item_table],
                          axis=0).astype(jnp.bfloat16)

    e1 = _propagate(src, dst, w, emb)
    e2 = _propagate(src, dst, w, e1)
    e3 = _propagate(src, dst, w, e2)

    u_sum = _user_gather(e1, e2, e3, user_ids)
    return _matmul(u_sum, e1[USERS:], e2[USERS:], e3[USERS:])
